# Initial kernel scaffold; baseline (speedup 1.0000x reference)
import jax
import jax.numpy as jnp
from jax.experimental import pallas as pl


def kernel(placeholder, table, labs):
    raise NotImplementedError("write your pallas kernel here")


# speedup vs baseline: 16.7156x; 16.7156x over previous
import functools

import jax
import jax.numpy as jnp
import numpy as np
from jax import lax
from jax.experimental import pallas as pl
from jax.experimental.pallas import tpu as pltpu
from jax.experimental.pallas import tpu_sc as plsc

IPC = 200
NUM_CLASSES = 100
CHANNEL, H, W = 3, 32, 32
N_PER_C = 10
DIM = CHANNEL * H * W
B = NUM_CLASSES * N_PER_C
ROWS = IPC * NUM_CLASSES

NC, NS = 2, 16
NW = NC * NS
B_PAD = 1024
BPW = B_PAD // NW


def _static_indices() -> np.ndarray:
    key = jax.random.key(42)
    parts = []
    for i in range(NUM_CLASSES):
        perm = jax.random.permutation(jax.random.fold_in(key, i), IPC)[:N_PER_C]
        parts.append(np.sort(np.asarray(perm)) + IPC * i)
    return np.concatenate(parts).astype(np.int32)


_INDICES = _static_indices()
_IDX_PAD = np.concatenate([_INDICES, np.zeros(B_PAD - B, np.int32)])

_mesh = plsc.VectorSubcoreMesh(core_axis_name="c", subcore_axis_name="s")


@functools.partial(
    pl.kernel,
    mesh=_mesh,
    out_type=(
        jax.ShapeDtypeStruct((B_PAD, DIM), jnp.float32),
        jax.ShapeDtypeStruct((B_PAD,), jnp.int32),
    ),
    scratch_types=[
        pltpu.VMEM((BPW,), jnp.int32),
        pltpu.VMEM((BPW, DIM), jnp.float32),
        pltpu.VMEM((BPW,), jnp.int32),
        pltpu.SemaphoreType.DMA,
    ],
)
def _gather_sc(table, idx, labs, out, labs_out, idx_v, rows_v, labs_v, sem):
    wid = lax.axis_index("s") * NC + lax.axis_index("c")
    base = wid * BPW
    pltpu.sync_copy(idx.at[pl.ds(base, BPW)], idx_v)
    pltpu.async_copy(table.at[idx_v], rows_v, sem).wait()
    pltpu.sync_copy(rows_v, out.at[pl.ds(base, BPW)])
    pltpu.async_copy(labs.at[idx_v], labs_v, sem).wait()
    pltpu.sync_copy(labs_v, labs_out.at[pl.ds(base, BPW)])


def kernel(placeholder, table, labs):
    idx_pad = jnp.asarray(_IDX_PAD)
    out, labs_out = _gather_sc(table, idx_pad, labs)
    imgs = out[:B].reshape(B, CHANNEL, H, W)
    indices = jnp.asarray(_INDICES)
    return (imgs, labs_out[:B], indices)


# speedup vs baseline: 20.9129x; 1.2511x over previous
import functools

import jax
import jax.numpy as jnp
import numpy as np
from jax import lax
from jax.experimental import pallas as pl
from jax.experimental.pallas import tpu as pltpu
from jax.experimental.pallas import tpu_sc as plsc

IPC = 200
NUM_CLASSES = 100
CHANNEL, H, W = 3, 32, 32
N_PER_C = 10
DIM = CHANNEL * H * W
B = NUM_CLASSES * N_PER_C
ROWS = IPC * NUM_CLASSES

NC, NS = 2, 16
NW = NC * NS
B_PAD = 1024
BPW = B_PAD // NW


def _static_indices() -> np.ndarray:
    key = jax.random.key(42)
    parts = []
    for i in range(NUM_CLASSES):
        perm = jax.random.permutation(jax.random.fold_in(key, i), IPC)[:N_PER_C]
        parts.append(np.sort(np.asarray(perm)) + IPC * i)
    return np.concatenate(parts).astype(np.int32)


_INDICES = _static_indices()
_IDX_PAD = np.concatenate([_INDICES, np.zeros(B_PAD - B, np.int32)])

_mesh = plsc.VectorSubcoreMesh(core_axis_name="c", subcore_axis_name="s")


CHUNK = 8
NCHUNK = BPW // CHUNK


@functools.partial(
    pl.kernel,
    mesh=_mesh,
    out_type=(
        jax.ShapeDtypeStruct((B, DIM), jnp.float32),
        jax.ShapeDtypeStruct((B,), jnp.int32),
    ),
    scratch_types=[
        pltpu.VMEM((BPW,), jnp.int32),
        pltpu.VMEM((BPW, DIM), jnp.float32),
        pltpu.VMEM((BPW,), jnp.int32),
        pltpu.SemaphoreType.DMA,
    ],
)
def _gather_sc(table, idx, labs, out, labs_out, idx_v, rows_v, labs_v, sem):
    wid = lax.axis_index("s") * NC + lax.axis_index("c")
    base = wid * BPW
    pltpu.sync_copy(idx.at[pl.ds(base, BPW)], idx_v)
    pltpu.async_copy(table.at[idx_v], rows_v, sem).wait()
    pltpu.async_copy(labs.at[idx_v], labs_v, sem).wait()
    for k in range(NCHUNK):
        off = base + k * CHUNK

        @pl.when(off < B)
        def _():
            pltpu.sync_copy(rows_v.at[pl.ds(k * CHUNK, CHUNK)],
                            out.at[pl.ds(off, CHUNK)])
            pltpu.sync_copy(labs_v.at[pl.ds(k * CHUNK, CHUNK)],
                            labs_out.at[pl.ds(off, CHUNK)])


def kernel(placeholder, table, labs):
    idx_pad = jnp.asarray(_IDX_PAD)
    out, labs_out = _gather_sc(table, idx_pad, labs)
    imgs = out.reshape(B, CHANNEL, H, W)
    indices = jnp.asarray(_INDICES)
    return (imgs, labs_out, indices)
